# baseline (speedup 1.0000x reference)
import functools

import jax
import jax.numpy as jnp
from jax import lax
from jax.experimental import pallas as pl
from jax.experimental.pallas import tpu as pltpu
from jax.experimental.pallas import tpu_sc as plsc

_NC, _NS = 2, 16
_NW = _NC * _NS
_C = 80
_R = 25


def _deg_body(dst4d, zeros1d, cnt_out, idx_v, ones_v, acc_sh):
    cid = lax.axis_index("c")
    sid = lax.axis_index("s")
    wid = cid * _NS + sid
    npad = acc_sh.shape[0]
    seg = npad // _NS
    for l in range(_C // 16):
        ones_v[pl.ds(l * 16, 16)] = jnp.ones((16,), jnp.float32)
    pltpu.sync_copy(zeros1d.at[pl.ds(sid * seg, seg)],
                    acc_sh.at[pl.ds(sid * seg, seg)])
    plsc.subcore_barrier()
    nblk = dst4d.shape[1]

    def blk(b, carry):
        pltpu.sync_copy(dst4d.at[wid, b], idx_v)

        def row(j, c2):
            pltpu.sync_copy(ones_v, acc_sh.at[idx_v.at[j]], add=True)
            return c2

        lax.fori_loop(0, _R, row, 0)
        return carry

    lax.fori_loop(0, nblk, blk, 0)
    plsc.subcore_barrier()
    pltpu.sync_copy(acc_sh.at[pl.ds(sid * seg, seg)],
                    cnt_out.at[cid, pl.ds(sid * seg, seg)])


_NSETS = 8


def _layer_body(g_hbm, src4d, dst4d, out_hbm,
                idxs_v, idxd_v, rows_v, acc_sh, *sems):
    gsems = sems[:_NSETS]
    ssems = sems[_NSETS:]
    cid = lax.axis_index("c")
    sid = lax.axis_index("s")
    wid = cid * _NS + sid
    n = acc_sh.shape[0]
    seg = n // _NS

    pltpu.sync_copy(g_hbm.at[pl.ds(sid * seg, seg)],
                    acc_sh.at[pl.ds(sid * seg, seg)])

    nblk = src4d.shape[1]
    for b in range(nblk):
        pltpu.sync_copy(src4d.at[wid, b], idxs_v.at[b])
        pltpu.sync_copy(dst4d.at[wid, b], idxd_v.at[b])
    plsc.subcore_barrier()

    ngrp = nblk * _R

    def fire_gather(g, p):
        pltpu.async_copy(g_hbm.at[idxs_v.at[g // _R, g % _R]],
                         rows_v.at[p], gsems[p])

    def drain_gather(p):
        pltpu.make_async_copy(g_hbm.at[idxs_v.at[0, 0]],
                              rows_v.at[p], gsems[p]).wait()

    def fire_scatters(g, p):
        pltpu.async_copy(rows_v.at[p],
                         acc_sh.at[idxd_v.at[g // _R, g % _R]],
                         ssems[p], add=True)

    def drain_scatters(p):
        pltpu.make_async_copy(rows_v.at[p],
                              acc_sh.at[idxd_v.at[0, 0]],
                              ssems[p]).wait()

    def grp(g, carry):
        for p in range(_NSETS):
            @pl.when(g % _NSETS == p)
            def _(p=p):
                @pl.when(g >= _NSETS)
                def _(p=p):
                    drain_scatters(p)
                fire_gather(g, p)
                pm1 = (p + _NSETS - 1) % _NSETS

                @pl.when(g >= 1)
                def _(p=p, pm1=pm1):
                    drain_gather(pm1)
                    fire_scatters(g - 1, pm1)
        return carry

    lax.fori_loop(0, ngrp, grp, 0)
    plast = (ngrp - 1) % _NSETS
    drain_gather(plast)
    fire_scatters(ngrp - 1, plast)
    for d in range(_NSETS - 1):
        drain_scatters((ngrp - 2 - d) % _NSETS)
    drain_scatters(plast)
    plsc.subcore_barrier()
    pltpu.sync_copy(acc_sh.at[pl.ds(sid * seg, seg)],
                    out_hbm.at[cid, pl.ds(sid * seg, seg)])


def _deg_call(dst4d, zeros1d, npad):
    mesh = plsc.VectorSubcoreMesh(core_axis_name="c", subcore_axis_name="s")
    f = functools.partial(
        pl.kernel,
        mesh=mesh,
        compiler_params=pltpu.CompilerParams(use_tc_tiling_on_sc=False),
        out_type=jax.ShapeDtypeStruct((_NC, npad), jnp.float32),
        scratch_types=[
            pltpu.VMEM((_R, _C), jnp.int32),
            pltpu.VMEM((_C,), jnp.float32),
            pltpu.VMEM_SHARED((npad,), jnp.float32),
        ],
    )(_deg_body)
    return f(dst4d, zeros1d)


def _layer_call(g, src4d, dst4d):
    npad, hid = g.shape
    mesh = plsc.VectorSubcoreMesh(core_axis_name="c", subcore_axis_name="s")
    f = functools.partial(
        pl.kernel,
        mesh=mesh,
        compiler_params=pltpu.CompilerParams(use_tc_tiling_on_sc=False),
        out_type=jax.ShapeDtypeStruct((_NC, npad, hid), jnp.float32),
        scratch_types=[
            pltpu.VMEM((src4d.shape[1], _R, _C), jnp.int32),
            pltpu.VMEM((src4d.shape[1], _R, _C), jnp.int32),
            pltpu.VMEM((_NSETS, _C, hid), jnp.float32),
            pltpu.VMEM_SHARED((npad, hid), jnp.float32),
        ] + [pltpu.SemaphoreType.DMA] * (2 * _NSETS),
    )(_layer_body)
    return f(g, src4d, dst4d)


def _tc1_body(ca_ref, cb_ref, x_ref, w_ref, g_ref, dinv_ref):
    c = ca_ref[...] + cb_ref[...] + 1.0
    dinv = lax.rsqrt(c)
    h = jnp.dot(x_ref[...], w_ref[...], preferred_element_type=jnp.float32)
    g_ref[...] = h * dinv
    dinv_ref[...] = dinv


def _tc2_body(aa_ref, ab_ref, g1_ref, dinv_ref, b1_ref, w2_ref, g2_ref):
    dinv = dinv_ref[...]
    s = aa_ref[...] + ab_ref[...] - g1_ref[...]
    z1 = jnp.maximum(s * dinv + b1_ref[...], 0.0)
    g2_ref[...] = jnp.dot(z1, w2_ref[...],
                          preferred_element_type=jnp.float32) * dinv


def _tc3_body(aa_ref, ab_ref, g2_ref, dinv_ref, b2_ref, z_ref):
    s = aa_ref[...] + ab_ref[...] - g2_ref[...]
    z_ref[...] = s * dinv_ref[...] + b2_ref[...]


_TB = 1000


def _tc1_call(ca, cb, x, w1):
    n, ind = x.shape
    hid = w1.shape[1]
    return pl.pallas_call(
        _tc1_body,
        grid=(n // _TB,),
        in_specs=[
            pl.BlockSpec((_TB, 1), lambda i: (i, 0)),
            pl.BlockSpec((_TB, 1), lambda i: (i, 0)),
            pl.BlockSpec((_TB, ind), lambda i: (i, 0)),
            pl.BlockSpec((ind, hid), lambda i: (0, 0)),
        ],
        out_specs=[
            pl.BlockSpec((_TB, hid), lambda i: (i, 0)),
            pl.BlockSpec((_TB, 1), lambda i: (i, 0)),
        ],
        out_shape=[
            jax.ShapeDtypeStruct((n, hid), jnp.float32),
            jax.ShapeDtypeStruct((n, 1), jnp.float32),
        ],
    )(ca, cb, x, w1)


def _tc2_call(aa, ab, g1, dinv, b1, w2):
    n, hid = aa.shape
    return pl.pallas_call(
        _tc2_body,
        grid=(n // _TB,),
        in_specs=[
            pl.BlockSpec((_TB, hid), lambda i: (i, 0)),
            pl.BlockSpec((_TB, hid), lambda i: (i, 0)),
            pl.BlockSpec((_TB, hid), lambda i: (i, 0)),
            pl.BlockSpec((_TB, 1), lambda i: (i, 0)),
            pl.BlockSpec((1, hid), lambda i: (0, 0)),
            pl.BlockSpec((hid, hid), lambda i: (0, 0)),
        ],
        out_specs=pl.BlockSpec((_TB, hid), lambda i: (i, 0)),
        out_shape=jax.ShapeDtypeStruct((n, hid), jnp.float32),
    )(aa, ab, g1, dinv, b1, w2)


def _tc3_call(aa, ab, g2, dinv, b2):
    n, hid = aa.shape
    return pl.pallas_call(
        _tc3_body,
        grid=(n // _TB,),
        in_specs=[
            pl.BlockSpec((_TB, hid), lambda i: (i, 0)),
            pl.BlockSpec((_TB, hid), lambda i: (i, 0)),
            pl.BlockSpec((_TB, hid), lambda i: (i, 0)),
            pl.BlockSpec((_TB, 1), lambda i: (i, 0)),
            pl.BlockSpec((1, hid), lambda i: (0, 0)),
        ],
        out_specs=pl.BlockSpec((_TB, hid), lambda i: (i, 0)),
        out_shape=jax.ShapeDtypeStruct((n, hid), jnp.float32),
    )(aa, ab, g2, dinv, b2)


def kernel(x, edge_index, W1, b1, W2, b2):
    n, _ = x.shape
    hid = W1.shape[1]
    src4d = edge_index[0].reshape(_NW, -1, _R, _C)
    dst4d = edge_index[1].reshape(_NW, -1, _R, _C)
    blk = 8 * _NS
    npad = ((n + blk - 1) // blk) * blk
    zeros1d = jnp.zeros((npad,), jnp.float32)

    cnt = _deg_call(dst4d, zeros1d, npad)
    ca = cnt[0, :n, None]
    cb = cnt[1, :n, None]
    g1, dinv = _tc1_call(ca, cb, x, W1)
    acc1 = _layer_call(g1, src4d, dst4d)
    g2 = _tc2_call(acc1[0], acc1[1], g1, dinv, b1.reshape(1, -1), W2)
    acc2 = _layer_call(g2, src4d, dst4d)
    z = _tc3_call(acc2[0], acc2[1], g2, dinv, b2.reshape(1, -1))
    return z


# speedup vs baseline: 1.1111x; 1.1111x over previous
import functools

import jax
import jax.numpy as jnp
from jax import lax
from jax.experimental import pallas as pl
from jax.experimental.pallas import tpu as pltpu
from jax.experimental.pallas import tpu_sc as plsc

_NC, _NS = 2, 16
_NW = _NC * _NS
_C = 80
_R = 25


def _deg_body(dst4d, zeros1d, cnt_out, idx_v, ones_v, acc_sh):
    cid = lax.axis_index("c")
    sid = lax.axis_index("s")
    wid = cid * _NS + sid
    npad = acc_sh.shape[0]
    seg = npad // _NS
    for l in range(_C // 16):
        ones_v[pl.ds(l * 16, 16)] = jnp.ones((16,), jnp.float32)
    pltpu.sync_copy(zeros1d.at[pl.ds(sid * seg, seg)],
                    acc_sh.at[pl.ds(sid * seg, seg)])
    plsc.subcore_barrier()
    nblk = dst4d.shape[1]

    def blk(b, carry):
        pltpu.sync_copy(dst4d.at[wid, b], idx_v)

        def row(j, c2):
            pltpu.sync_copy(ones_v, acc_sh.at[idx_v.at[j]], add=True)
            return c2

        lax.fori_loop(0, _R, row, 0)
        return carry

    lax.fori_loop(0, nblk, blk, 0)
    plsc.subcore_barrier()
    pltpu.sync_copy(acc_sh.at[pl.ds(sid * seg, seg)],
                    cnt_out.at[cid, pl.ds(sid * seg, seg)])


_NSETS = 8


def _layer_body(g_hbm, src4d, dst4d, out_hbm,
                idxs_v, idxd_v, rows_v, acc_sh, *sems):
    gsems = sems[:_NSETS]
    ssems = sems[_NSETS:]
    cid = lax.axis_index("c")
    sid = lax.axis_index("s")
    wid = cid * _NS + sid
    n = acc_sh.shape[0]
    seg = n // _NS

    pltpu.sync_copy(g_hbm.at[pl.ds(sid * seg, seg)],
                    acc_sh.at[pl.ds(sid * seg, seg)])

    nblk = src4d.shape[1]
    for b in range(nblk):
        pltpu.sync_copy(src4d.at[wid, b], idxs_v.at[b])
        pltpu.sync_copy(dst4d.at[wid, b], idxd_v.at[b])
    plsc.subcore_barrier()

    ngrp = nblk * _R

    def fire_gather(g, p):
        pltpu.async_copy(g_hbm.at[idxs_v.at[g // _R, g % _R]],
                         rows_v.at[p], gsems[p])

    def drain_gather(p):
        pltpu.make_async_copy(g_hbm.at[idxs_v.at[0, 0]],
                              rows_v.at[p], gsems[p]).wait()

    def fire_scatters(g, p):
        pltpu.async_copy(rows_v.at[p],
                         acc_sh.at[idxd_v.at[g // _R, g % _R]],
                         ssems[p], add=True)

    def drain_scatters(p):
        pltpu.make_async_copy(rows_v.at[p],
                              acc_sh.at[idxd_v.at[0, 0]],
                              ssems[p]).wait()

    def grp(g, carry):
        for p in range(_NSETS):
            @pl.when(g % _NSETS == p)
            def _(p=p):
                @pl.when(g >= _NSETS)
                def _(p=p):
                    drain_scatters(p)
                fire_gather(g, p)
                pm1 = (p + _NSETS - 1) % _NSETS

                @pl.when(g >= 1)
                def _(p=p, pm1=pm1):
                    drain_gather(pm1)
                    fire_scatters(g - 1, pm1)
        return carry

    lax.fori_loop(0, ngrp, grp, 0)
    plast = (ngrp - 1) % _NSETS
    drain_gather(plast)
    fire_scatters(ngrp - 1, plast)
    for d in range(_NSETS - 1):
        drain_scatters((ngrp - 2 - d) % _NSETS)
    drain_scatters(plast)
    plsc.subcore_barrier()
    pltpu.sync_copy(acc_sh.at[pl.ds(sid * seg, seg)],
                    out_hbm.at[cid, pl.ds(sid * seg, seg)])


def _deg_call(dst4d, zeros1d, npad):
    mesh = plsc.VectorSubcoreMesh(core_axis_name="c", subcore_axis_name="s")
    f = functools.partial(
        pl.kernel,
        mesh=mesh,
        compiler_params=pltpu.CompilerParams(use_tc_tiling_on_sc=False),
        out_type=jax.ShapeDtypeStruct((_NC, npad), jnp.float32),
        scratch_types=[
            pltpu.VMEM((_R, _C), jnp.int32),
            pltpu.VMEM((_C,), jnp.float32),
            pltpu.VMEM_SHARED((npad,), jnp.float32),
        ],
    )(_deg_body)
    return f(dst4d, zeros1d)


def _layer_call(g, src4d, dst4d):
    npad, hid = g.shape
    mesh = plsc.VectorSubcoreMesh(core_axis_name="c", subcore_axis_name="s")
    f = functools.partial(
        pl.kernel,
        mesh=mesh,
        compiler_params=pltpu.CompilerParams(use_tc_tiling_on_sc=False),
        out_type=jax.ShapeDtypeStruct((_NC, npad, hid), jnp.float32),
        scratch_types=[
            pltpu.VMEM((src4d.shape[1], _R, _C), jnp.int32),
            pltpu.VMEM((src4d.shape[1], _R, _C), jnp.int32),
            pltpu.VMEM((_NSETS, _C, hid), jnp.float32),
            pltpu.VMEM_SHARED((npad, hid), jnp.float32),
        ] + [pltpu.SemaphoreType.DMA] * (2 * _NSETS),
    )(_layer_body)
    return f(g, src4d, dst4d)


def _tc1_body(cnt_ref, x_ref, w_ref, ones_ref, g_ref, dinv_ref):
    n = x_ref.shape[0]
    crow = (cnt_ref[0] + cnt_ref[1] + 1.0).reshape(1, -1)
    drow = lax.rsqrt(crow)
    dinv = lax.dot_general(drow, ones_ref[...], (((0,), (0,)), ((), ())),
                           preferred_element_type=jnp.float32)[:n]
    h = jnp.dot(x_ref[...], w_ref[...], preferred_element_type=jnp.float32)
    g_ref[...] = h * dinv
    dinv_ref[...] = dinv


def _tc2_body(aa_ref, ab_ref, g1_ref, dinv_ref, b1_ref, w2_ref, g2_ref):
    dinv = dinv_ref[...]
    s = aa_ref[0] + ab_ref[0] - g1_ref[...]
    z1 = jnp.maximum(s * dinv + b1_ref[...], 0.0)
    g2_ref[...] = jnp.dot(z1, w2_ref[...],
                          preferred_element_type=jnp.float32) * dinv


def _tc3_body(aa_ref, ab_ref, g2_ref, dinv_ref, b2_ref, z_ref):
    s = aa_ref[0] + ab_ref[0] - g2_ref[...]
    z_ref[...] = s * dinv_ref[...] + b2_ref[...]


_TB = 2000


def _tc1_call(cnt, x, w1):
    n, ind = x.shape
    hid = w1.shape[1]
    ones = jnp.ones((1, hid), jnp.float32)
    return pl.pallas_call(
        _tc1_body,
        out_shape=[
            jax.ShapeDtypeStruct((n, hid), jnp.float32),
            jax.ShapeDtypeStruct((n, hid), jnp.float32),
        ],
    )(cnt, x, w1, ones)


def _tc2_call(acc, g1, dinv, b1, w2):
    _, n, hid = acc.shape
    return pl.pallas_call(
        _tc2_body,
        grid=(n // _TB,),
        in_specs=[
            pl.BlockSpec((1, _TB, hid), lambda i: (0, i, 0)),
            pl.BlockSpec((1, _TB, hid), lambda i: (1, i, 0)),
            pl.BlockSpec((_TB, hid), lambda i: (i, 0)),
            pl.BlockSpec((_TB, hid), lambda i: (i, 0)),
            pl.BlockSpec((1, hid), lambda i: (0, 0)),
            pl.BlockSpec((hid, hid), lambda i: (0, 0)),
        ],
        out_specs=pl.BlockSpec((_TB, hid), lambda i: (i, 0)),
        out_shape=jax.ShapeDtypeStruct((n, hid), jnp.float32),
    )(acc, acc, g1, dinv, b1, w2)


def _tc3_call(acc, g2, dinv, b2):
    _, n, hid = acc.shape
    return pl.pallas_call(
        _tc3_body,
        grid=(n // _TB,),
        in_specs=[
            pl.BlockSpec((1, _TB, hid), lambda i: (0, i, 0)),
            pl.BlockSpec((1, _TB, hid), lambda i: (1, i, 0)),
            pl.BlockSpec((_TB, hid), lambda i: (i, 0)),
            pl.BlockSpec((_TB, hid), lambda i: (i, 0)),
            pl.BlockSpec((1, hid), lambda i: (0, 0)),
        ],
        out_specs=pl.BlockSpec((_TB, hid), lambda i: (i, 0)),
        out_shape=jax.ShapeDtypeStruct((n, hid), jnp.float32),
    )(acc, acc, g2, dinv, b2)


def kernel(x, edge_index, W1, b1, W2, b2):
    n, _ = x.shape
    hid = W1.shape[1]
    src4d = edge_index[0].reshape(_NW, -1, _R, _C)
    dst4d = edge_index[1].reshape(_NW, -1, _R, _C)
    blk = 8 * _NS
    npad = ((n + blk - 1) // blk) * blk
    zeros1d = jnp.zeros((npad,), jnp.float32)

    cnt = _deg_call(dst4d, zeros1d, npad)
    g1, dinv = _tc1_call(cnt, x, W1)
    acc1 = _layer_call(g1, src4d, dst4d)
    g2 = _tc2_call(acc1, g1, dinv, b1.reshape(1, -1), W2)
    acc2 = _layer_call(g2, src4d, dst4d)
    z = _tc3_call(acc2, g2, dinv, b2.reshape(1, -1))
    return z


# speedup vs baseline: 1.1494x; 1.0345x over previous
import functools

import jax
import jax.numpy as jnp
from jax import lax
from jax.experimental import pallas as pl
from jax.experimental.pallas import tpu as pltpu
from jax.experimental.pallas import tpu_sc as plsc

_NC, _NS = 2, 16
_NW = _NC * _NS
_C = 80
_R = 25


def _deg_body(dst4d, zeros1d, cnt_out, idx_v, ones_v, acc_sh):
    cid = lax.axis_index("c")
    sid = lax.axis_index("s")
    wid = cid * _NS + sid
    npad = acc_sh.shape[0]
    seg = npad // _NS
    for l in range(_C // 16):
        ones_v[pl.ds(l * 16, 16)] = jnp.ones((16,), jnp.float32)
    pltpu.sync_copy(zeros1d.at[pl.ds(sid * seg, seg)],
                    acc_sh.at[pl.ds(sid * seg, seg)])
    plsc.subcore_barrier()
    nblk = dst4d.shape[1]

    def blk(b, carry):
        pltpu.sync_copy(dst4d.at[wid, b], idx_v)

        def row(j, c2):
            pltpu.sync_copy(ones_v, acc_sh.at[idx_v.at[j]], add=True)
            return c2

        lax.fori_loop(0, _R, row, 0)
        return carry

    lax.fori_loop(0, nblk, blk, 0)
    plsc.subcore_barrier()
    pltpu.sync_copy(acc_sh.at[pl.ds(sid * seg, seg)],
                    cnt_out.at[cid, pl.ds(sid * seg, seg)])


_NSETS = 8


def _layer_body(g_hbm, src4d, dst4d, out_hbm,
                idxs_v, idxd_v, rows_v, acc_sh, *sems):
    gsems = sems[:_NSETS]
    ssems = sems[_NSETS:]
    cid = lax.axis_index("c")
    sid = lax.axis_index("s")
    wid = cid * _NS + sid
    n = acc_sh.shape[0]
    seg = n // _NS

    pltpu.sync_copy(g_hbm.at[pl.ds(sid * seg, seg)],
                    acc_sh.at[pl.ds(sid * seg, seg)])

    nblk = src4d.shape[1]
    for b in range(nblk):
        pltpu.sync_copy(src4d.at[wid, b], idxs_v.at[b])
        pltpu.sync_copy(dst4d.at[wid, b], idxd_v.at[b])
    plsc.subcore_barrier()

    ngrp = nblk * _R

    def fire_gather(g, p):
        pltpu.async_copy(g_hbm.at[idxs_v.at[g // _R, g % _R]],
                         rows_v.at[p], gsems[p])

    def drain_gather(p):
        pltpu.make_async_copy(g_hbm.at[idxs_v.at[0, 0]],
                              rows_v.at[p], gsems[p]).wait()

    def fire_scatters(g, p):
        pltpu.async_copy(rows_v.at[p],
                         acc_sh.at[idxd_v.at[g // _R, g % _R]],
                         ssems[p], add=True)

    def drain_scatters(p):
        pltpu.make_async_copy(rows_v.at[p],
                              acc_sh.at[idxd_v.at[0, 0]],
                              ssems[p]).wait()

    def grp(g, carry):
        for p in range(_NSETS):
            @pl.when(g % _NSETS == p)
            def _(p=p):
                @pl.when(g >= _NSETS)
                def _(p=p):
                    drain_scatters(p)
                fire_gather(g, p)
                pm1 = (p + _NSETS - 1) % _NSETS

                @pl.when(g >= 1)
                def _(p=p, pm1=pm1):
                    drain_gather(pm1)
                    fire_scatters(g - 1, pm1)
        return carry

    lax.fori_loop(0, ngrp, grp, 0)
    plast = (ngrp - 1) % _NSETS
    drain_gather(plast)
    fire_scatters(ngrp - 1, plast)
    for d in range(_NSETS - 1):
        drain_scatters((ngrp - 2 - d) % _NSETS)
    drain_scatters(plast)
    plsc.subcore_barrier()
    pltpu.sync_copy(acc_sh.at[pl.ds(sid * seg, seg)],
                    out_hbm.at[cid, pl.ds(sid * seg, seg)])


def _deg_call(dst4d, zeros1d, npad):
    mesh = plsc.VectorSubcoreMesh(core_axis_name="c", subcore_axis_name="s")
    f = functools.partial(
        pl.kernel,
        mesh=mesh,
        compiler_params=pltpu.CompilerParams(use_tc_tiling_on_sc=False),
        out_type=jax.ShapeDtypeStruct((_NC, npad), jnp.float32),
        scratch_types=[
            pltpu.VMEM((_R, _C), jnp.int32),
            pltpu.VMEM((_C,), jnp.float32),
            pltpu.VMEM_SHARED((npad,), jnp.float32),
        ],
    )(_deg_body)
    return f(dst4d, zeros1d)


def _layer_call(g, src4d, dst4d):
    npad, hid = g.shape
    mesh = plsc.VectorSubcoreMesh(core_axis_name="c", subcore_axis_name="s")
    f = functools.partial(
        pl.kernel,
        mesh=mesh,
        compiler_params=pltpu.CompilerParams(use_tc_tiling_on_sc=False),
        out_type=jax.ShapeDtypeStruct((_NC, npad, hid), jnp.float32),
        scratch_types=[
            pltpu.VMEM((src4d.shape[1], _R, _C), jnp.int32),
            pltpu.VMEM((src4d.shape[1], _R, _C), jnp.int32),
            pltpu.VMEM((_NSETS, _C, hid), jnp.float32),
            pltpu.VMEM_SHARED((npad, hid), jnp.float32),
        ] + [pltpu.SemaphoreType.DMA] * (2 * _NSETS),
    )(_layer_body)
    return f(g, src4d, dst4d)


def _tc1_body(cnt_ref, x_ref, w_ref, ones_ref, g_ref, dinv_ref):
    n = x_ref.shape[0]
    crow = (cnt_ref[0] + cnt_ref[1] + 1.0).reshape(1, -1)
    drow = lax.rsqrt(crow)
    dinv = lax.dot_general(drow, ones_ref[...], (((0,), (0,)), ((), ())),
                           precision=lax.Precision.HIGHEST,
                           preferred_element_type=jnp.float32)[:n]
    h = jnp.dot(x_ref[...], w_ref[...], preferred_element_type=jnp.float32)
    g_ref[...] = h * dinv
    dinv_ref[...] = dinv


def _tc2_body(aa_ref, ab_ref, g1_ref, dinv_ref, b1_ref, w2_ref, g2_ref):
    dinv = dinv_ref[...]
    s = aa_ref[0] + ab_ref[0] - g1_ref[...]
    z1 = jnp.maximum(s * dinv + b1_ref[...], 0.0)
    g2_ref[...] = jnp.dot(z1, w2_ref[...],
                          preferred_element_type=jnp.float32) * dinv


def _tc3_body(aa_ref, ab_ref, g2_ref, dinv_ref, b2_ref, z_ref):
    s = aa_ref[0] + ab_ref[0] - g2_ref[...]
    z_ref[...] = s * dinv_ref[...] + b2_ref[...]


_TB = 2000


def _detile_body(e_ref, s_ref, d_ref):
    s_ref[...] = e_ref[0]
    d_ref[...] = e_ref[1]


def _detile_call(edge_index):
    e = edge_index.shape[1]
    return pl.pallas_call(
        _detile_body,
        out_shape=[jax.ShapeDtypeStruct((e,), jnp.int32),
                   jax.ShapeDtypeStruct((e,), jnp.int32)],
    )(edge_index)


def _tc1_call(cnt, x, w1):
    n, ind = x.shape
    hid = w1.shape[1]
    ones = jnp.ones((1, hid), jnp.float32)
    return pl.pallas_call(
        _tc1_body,
        out_shape=[
            jax.ShapeDtypeStruct((n, hid), jnp.float32),
            jax.ShapeDtypeStruct((n, hid), jnp.float32),
        ],
    )(cnt, x, w1, ones)


def _tc2_call(acc, g1, dinv, b1, w2):
    _, n, hid = acc.shape
    return pl.pallas_call(
        _tc2_body,
        grid=(n // _TB,),
        in_specs=[
            pl.BlockSpec((1, _TB, hid), lambda i: (0, i, 0)),
            pl.BlockSpec((1, _TB, hid), lambda i: (1, i, 0)),
            pl.BlockSpec((_TB, hid), lambda i: (i, 0)),
            pl.BlockSpec((_TB, hid), lambda i: (i, 0)),
            pl.BlockSpec((1, hid), lambda i: (0, 0)),
            pl.BlockSpec((hid, hid), lambda i: (0, 0)),
        ],
        out_specs=pl.BlockSpec((_TB, hid), lambda i: (i, 0)),
        out_shape=jax.ShapeDtypeStruct((n, hid), jnp.float32),
    )(acc, acc, g1, dinv, b1, w2)


def _tc3_call(acc, g2, dinv, b2):
    _, n, hid = acc.shape
    return pl.pallas_call(
        _tc3_body,
        grid=(n // _TB,),
        in_specs=[
            pl.BlockSpec((1, _TB, hid), lambda i: (0, i, 0)),
            pl.BlockSpec((1, _TB, hid), lambda i: (1, i, 0)),
            pl.BlockSpec((_TB, hid), lambda i: (i, 0)),
            pl.BlockSpec((_TB, hid), lambda i: (i, 0)),
            pl.BlockSpec((1, hid), lambda i: (0, 0)),
        ],
        out_specs=pl.BlockSpec((_TB, hid), lambda i: (i, 0)),
        out_shape=jax.ShapeDtypeStruct((n, hid), jnp.float32),
    )(acc, acc, g2, dinv, b2)


def kernel(x, edge_index, W1, b1, W2, b2):
    n, _ = x.shape
    hid = W1.shape[1]
    src1d, dst1d = _detile_call(edge_index)
    src4d = src1d.reshape(_NW, -1, _R, _C)
    dst4d = dst1d.reshape(_NW, -1, _R, _C)
    blk = 8 * _NS
    npad = ((n + blk - 1) // blk) * blk
    zeros1d = jnp.zeros((npad,), jnp.float32)

    cnt = _deg_call(dst4d, zeros1d, npad)
    g1, dinv = _tc1_call(cnt, x, W1)
    acc1 = _layer_call(g1, src4d, dst4d)
    g2 = _tc2_call(acc1, g1, dinv, b1.reshape(1, -1), W2)
    acc2 = _layer_call(g2, src4d, dst4d)
    z = _tc3_call(acc2, g2, dinv, b2.reshape(1, -1))
    return z


# speedup vs baseline: 1.2418x; 1.0804x over previous
import functools

import jax
import jax.numpy as jnp
from jax import lax
from jax.experimental import pallas as pl
from jax.experimental.pallas import tpu as pltpu
from jax.experimental.pallas import tpu_sc as plsc

_NC, _NS = 2, 16
_NW = _NC * _NS
_C = 80
_R = 25


def _deg_body(dst4d, zeros1d, cnt_out, idx_v, ones_v, acc_sh):
    cid = lax.axis_index("c")
    sid = lax.axis_index("s")
    wid = cid * _NS + sid
    npad = acc_sh.shape[0]
    seg = npad // _NS
    for l in range(_C // 16):
        ones_v[pl.ds(l * 16, 16)] = jnp.ones((16,), jnp.float32)
    pltpu.sync_copy(zeros1d.at[pl.ds(sid * seg, seg)],
                    acc_sh.at[pl.ds(sid * seg, seg)])
    plsc.subcore_barrier()
    nblk = dst4d.shape[1]

    def blk(b, carry):
        pltpu.sync_copy(dst4d.at[wid, b], idx_v)

        def row(j, c2):
            pltpu.sync_copy(ones_v, acc_sh.at[idx_v.at[j]], add=True)
            return c2

        lax.fori_loop(0, _R, row, 0)
        return carry

    lax.fori_loop(0, nblk, blk, 0)
    plsc.subcore_barrier()
    pltpu.sync_copy(acc_sh.at[pl.ds(sid * seg, seg)],
                    cnt_out.at[cid, pl.ds(sid * seg, seg)])


_NSETS = 4


def _layer_body(g_hbm, src4d, dst4d, out_hbm,
                idxs_v, idxd_v, rows_v, acc_sh, g_sh, *sems):
    gsems = sems[:_NSETS]
    ssems = sems[_NSETS:]
    cid = lax.axis_index("c")
    sid = lax.axis_index("s")
    wid = cid * _NS + sid
    n = acc_sh.shape[0]
    seg = n // _NS

    pltpu.sync_copy(g_hbm.at[pl.ds(sid * seg, seg)],
                    acc_sh.at[pl.ds(sid * seg, seg)])
    pltpu.sync_copy(g_hbm.at[pl.ds(sid * seg, seg)],
                    g_sh.at[pl.ds(sid * seg, seg)])

    nblk = src4d.shape[1]
    for b in range(nblk):
        pltpu.sync_copy(src4d.at[wid, b], idxs_v.at[b])
        pltpu.sync_copy(dst4d.at[wid, b], idxd_v.at[b])
    plsc.subcore_barrier()

    ngrp = nblk * _R

    def fire_gather(g, p):
        pltpu.async_copy(g_sh.at[idxs_v.at[g // _R, g % _R]],
                         rows_v.at[p], gsems[p])

    def drain_gather(p):
        pltpu.make_async_copy(g_sh.at[idxs_v.at[0, 0]],
                              rows_v.at[p], gsems[p]).wait()

    def fire_scatters(g, p):
        pltpu.async_copy(rows_v.at[p],
                         acc_sh.at[idxd_v.at[g // _R, g % _R]],
                         ssems[p], add=True)

    def drain_scatters(p):
        pltpu.make_async_copy(rows_v.at[p],
                              acc_sh.at[idxd_v.at[0, 0]],
                              ssems[p]).wait()

    def grp(g, carry):
        for p in range(_NSETS):
            @pl.when(g % _NSETS == p)
            def _(p=p):
                @pl.when(g >= _NSETS)
                def _(p=p):
                    drain_scatters(p)
                fire_gather(g, p)
                pm1 = (p + _NSETS - 1) % _NSETS

                @pl.when(g >= 1)
                def _(p=p, pm1=pm1):
                    drain_gather(pm1)
                    fire_scatters(g - 1, pm1)
        return carry

    lax.fori_loop(0, ngrp, grp, 0)
    plast = (ngrp - 1) % _NSETS
    drain_gather(plast)
    fire_scatters(ngrp - 1, plast)
    for d in range(_NSETS - 1):
        drain_scatters((ngrp - 2 - d) % _NSETS)
    drain_scatters(plast)
    plsc.subcore_barrier()
    pltpu.sync_copy(acc_sh.at[pl.ds(sid * seg, seg)],
                    out_hbm.at[cid, pl.ds(sid * seg, seg)])


def _deg_call(dst4d, zeros1d, npad):
    mesh = plsc.VectorSubcoreMesh(core_axis_name="c", subcore_axis_name="s")
    f = functools.partial(
        pl.kernel,
        mesh=mesh,
        compiler_params=pltpu.CompilerParams(use_tc_tiling_on_sc=False),
        out_type=jax.ShapeDtypeStruct((_NC, npad), jnp.float32),
        scratch_types=[
            pltpu.VMEM((_R, _C), jnp.int32),
            pltpu.VMEM((_C,), jnp.float32),
            pltpu.VMEM_SHARED((npad,), jnp.float32),
        ],
    )(_deg_body)
    return f(dst4d, zeros1d)


def _layer_call(g, src4d, dst4d):
    npad, hid = g.shape
    mesh = plsc.VectorSubcoreMesh(core_axis_name="c", subcore_axis_name="s")
    f = functools.partial(
        pl.kernel,
        mesh=mesh,
        compiler_params=pltpu.CompilerParams(use_tc_tiling_on_sc=False),
        out_type=jax.ShapeDtypeStruct((_NC, npad, hid), jnp.float32),
        scratch_types=[
            pltpu.VMEM((src4d.shape[1], _R, _C), jnp.int32),
            pltpu.VMEM((src4d.shape[1], _R, _C), jnp.int32),
            pltpu.VMEM((_NSETS, _C, hid), jnp.float32),
            pltpu.VMEM_SHARED((npad, hid), jnp.float32),
            pltpu.VMEM_SHARED((npad, hid), jnp.float32),
        ] + [pltpu.SemaphoreType.DMA] * (2 * _NSETS),
    )(_layer_body)
    return f(g, src4d, dst4d)


def _tc1_body(cnt_ref, x_ref, w_ref, ones_ref, g_ref, dinv_ref):
    n = x_ref.shape[0]
    crow = (cnt_ref[0] + cnt_ref[1] + 1.0).reshape(1, -1)
    drow = lax.rsqrt(crow)
    dinv = lax.dot_general(drow, ones_ref[...], (((0,), (0,)), ((), ())),
                           precision=lax.Precision.HIGHEST,
                           preferred_element_type=jnp.float32)[:n]
    h = jnp.dot(x_ref[...], w_ref[...], preferred_element_type=jnp.float32)
    g_ref[...] = h * dinv
    dinv_ref[...] = dinv


def _tc2_body(aa_ref, ab_ref, g1_ref, dinv_ref, b1_ref, w2_ref, g2_ref):
    dinv = dinv_ref[...]
    s = aa_ref[0] + ab_ref[0] - g1_ref[...]
    z1 = jnp.maximum(s * dinv + b1_ref[...], 0.0)
    g2_ref[...] = jnp.dot(z1, w2_ref[...],
                          preferred_element_type=jnp.float32) * dinv


def _tc3_body(aa_ref, ab_ref, g2_ref, dinv_ref, b2_ref, z_ref):
    s = aa_ref[0] + ab_ref[0] - g2_ref[...]
    z_ref[...] = s * dinv_ref[...] + b2_ref[...]


_TB = 2000


def _detile_body(e_ref, s_ref, d_ref):
    s_ref[...] = e_ref[0]
    d_ref[...] = e_ref[1]


def _detile_call(edge_index):
    e = edge_index.shape[1]
    return pl.pallas_call(
        _detile_body,
        out_shape=[jax.ShapeDtypeStruct((e,), jnp.int32),
                   jax.ShapeDtypeStruct((e,), jnp.int32)],
    )(edge_index)


def _tc1_call(cnt, x, w1):
    n, ind = x.shape
    hid = w1.shape[1]
    ones = jnp.ones((1, hid), jnp.float32)
    return pl.pallas_call(
        _tc1_body,
        out_shape=[
            jax.ShapeDtypeStruct((n, hid), jnp.float32),
            jax.ShapeDtypeStruct((n, hid), jnp.float32),
        ],
    )(cnt, x, w1, ones)


def _tc2_call(acc, g1, dinv, b1, w2):
    _, n, hid = acc.shape
    return pl.pallas_call(
        _tc2_body,
        grid=(n // _TB,),
        in_specs=[
            pl.BlockSpec((1, _TB, hid), lambda i: (0, i, 0)),
            pl.BlockSpec((1, _TB, hid), lambda i: (1, i, 0)),
            pl.BlockSpec((_TB, hid), lambda i: (i, 0)),
            pl.BlockSpec((_TB, hid), lambda i: (i, 0)),
            pl.BlockSpec((1, hid), lambda i: (0, 0)),
            pl.BlockSpec((hid, hid), lambda i: (0, 0)),
        ],
        out_specs=pl.BlockSpec((_TB, hid), lambda i: (i, 0)),
        out_shape=jax.ShapeDtypeStruct((n, hid), jnp.float32),
    )(acc, acc, g1, dinv, b1, w2)


def _tc3_call(acc, g2, dinv, b2):
    _, n, hid = acc.shape
    return pl.pallas_call(
        _tc3_body,
        grid=(n // _TB,),
        in_specs=[
            pl.BlockSpec((1, _TB, hid), lambda i: (0, i, 0)),
            pl.BlockSpec((1, _TB, hid), lambda i: (1, i, 0)),
            pl.BlockSpec((_TB, hid), lambda i: (i, 0)),
            pl.BlockSpec((_TB, hid), lambda i: (i, 0)),
            pl.BlockSpec((1, hid), lambda i: (0, 0)),
        ],
        out_specs=pl.BlockSpec((_TB, hid), lambda i: (i, 0)),
        out_shape=jax.ShapeDtypeStruct((n, hid), jnp.float32),
    )(acc, acc, g2, dinv, b2)


def kernel(x, edge_index, W1, b1, W2, b2):
    n, _ = x.shape
    hid = W1.shape[1]
    src1d, dst1d = _detile_call(edge_index)
    src4d = src1d.reshape(_NW, -1, _R, _C)
    dst4d = dst1d.reshape(_NW, -1, _R, _C)
    blk = 8 * _NS
    npad = ((n + blk - 1) // blk) * blk
    zeros1d = jnp.zeros((npad,), jnp.float32)

    cnt = _deg_call(dst4d, zeros1d, npad)
    g1, dinv = _tc1_call(cnt, x, W1)
    acc1 = _layer_call(g1, src4d, dst4d)
    g2 = _tc2_call(acc1, g1, dinv, b1.reshape(1, -1), W2)
    acc2 = _layer_call(g2, src4d, dst4d)
    z = _tc3_call(acc2, g2, dinv, b2.reshape(1, -1))
    return z
